# Initial kernel scaffold; baseline (speedup 1.0000x reference)
import jax
import jax.numpy as jnp
from jax.experimental import pallas as pl


def kernel(userIdx, itemIdx, adj_row, adj_col, adj_val, user_emb, item_emb, W1, b1, W2, b2, W3, b3):
    raise NotImplementedError("write your pallas kernel here")


# speedup vs baseline: 2.1370x; 2.1370x over previous
import jax
import jax.numpy as jnp
from jax import lax
from jax.experimental import pallas as pl
from jax.experimental.pallas import tpu as pltpu
from jax.experimental.pallas import tpu_sc as plsc

N_USERS = 25000
N_ITEMS = 25000
NP2 = 25088
EMB = 100
EPAD = 128
CW = 16
NCH = EPAD // CW
NC = 2
NS = 16
KB = 128
RPT = NP2 // NS

_SC_PARAMS = pltpu.CompilerParams(use_tc_tiling_on_sc=False)


def _layer(xc, ebat, eval_, nbh):
    mesh = plsc.VectorSubcoreMesh(core_axis_name="c", subcore_axis_name="s")
    nbt = nbh // NS

    def body(xc_ref, eb_ref, ev_ref, yc_ref,
             xsh, ysh, ebuf, valv, gbuf, zbuf, sem):
        c = lax.axis_index("c")
        s = lax.axis_index("s")
        r0 = s * RPT
        b0 = c * nbh + s * nbt

        def zb_body(k, _):
            zbuf[k, :] = jnp.zeros((CW,), jnp.float32)
            return 0
        lax.fori_loop(0, RPT, zb_body, 0, unroll=8)

        for cj in range(NCH):
            pltpu.sync_copy(xc_ref.at[1 - c, cj, pl.ds(r0, RPT), :],
                            xsh.at[pl.ds(r0, RPT), :])
            pltpu.sync_copy(zbuf, ysh.at[pl.ds(r0, RPT), :])
            plsc.subcore_barrier()

            def batch_body(b, _):
                pltpu.sync_copy(eb_ref.at[b0 + b], ebuf)
                pltpu.sync_copy(ev_ref.at[b0 + b], valv.at[pl.ds(0, KB)])
                pltpu.async_copy(xsh.at[ebuf.at[1]], gbuf, sem).wait()

                def mul_body(k, _):
                    v = valv[pl.ds(k, CW)][0]
                    gbuf[k, :] = gbuf[k, :] * v
                    return 0
                lax.fori_loop(0, KB, mul_body, 0, unroll=8)
                pltpu.sync_copy(gbuf, ysh.at[ebuf.at[0]], add=True)
                return 0
            lax.fori_loop(0, nbt, batch_body, 0)
            plsc.subcore_barrier()
            pltpu.sync_copy(ysh.at[pl.ds(r0, RPT), :],
                            yc_ref.at[c, cj, pl.ds(r0, RPT), :])

    out_t = jax.ShapeDtypeStruct((NC, NCH, NP2, CW), jnp.float32)
    scratch = [
        pltpu.VMEM_SHARED((NP2, CW), jnp.float32),
        pltpu.VMEM_SHARED((NP2, CW), jnp.float32),
        pltpu.VMEM((2, KB), jnp.int32),
        pltpu.VMEM((KB + CW,), jnp.float32),
        pltpu.VMEM((KB, CW), jnp.float32),
        pltpu.VMEM((RPT, CW), jnp.float32),
        pltpu.SemaphoreType.DMA,
    ]
    return pl.kernel(body, out_type=out_t, mesh=mesh, scratch_types=scratch,
                     compiler_params=_SC_PARAMS)(xc, ebat, eval_)


def _batch_gather(x0c, x1c, x2c, cidx, nb):
    mesh = plsc.VectorSubcoreMesh(core_axis_name="c", subcore_axis_name="s")
    bt = nb // (NC * NS)

    def body(x0c_ref, x1c_ref, x2c_ref, idx_ref, g0_ref, g1_ref, g2_ref,
             idxv, tmp, sem):
        c = lax.axis_index("c")
        s = lax.axis_index("s")
        wid = s * NC + c
        base = wid * bt
        side = wid // NS
        pltpu.sync_copy(idx_ref.at[pl.ds(base, bt)], idxv)
        for tbl_ref, out_ref in ((x0c_ref, g0_ref), (x1c_ref, g1_ref),
                                 (x2c_ref, g2_ref)):
            for cj in range(NCH):
                pltpu.async_copy(tbl_ref.at[side, cj].at[idxv], tmp,
                                 sem).wait()
                pltpu.sync_copy(
                    tmp, out_ref.at[pl.ds(base, bt), pl.ds(cj * CW, CW)])

    out_t = [jax.ShapeDtypeStruct((nb, EPAD), jnp.float32)] * 3
    scratch = [
        pltpu.VMEM((bt,), jnp.int32),
        pltpu.VMEM((bt, CW), jnp.float32),
        pltpu.SemaphoreType.DMA,
    ]
    return pl.kernel(body, out_type=out_t, mesh=mesh, scratch_types=scratch,
                     compiler_params=_SC_PARAMS)(x0c, x1c, x2c, cidx)


def _mlp_block(g0u, g0i, g1u, g1i, g2u, g2i, w1u, w1i, b1, w2, b2, w3, b3,
               out):
    third = jnp.float32(1.0 / 3.0)
    zu = (g0u[0] + g1u[0] + g2u[0]) * third
    zi = (g0i[0] + g1i[0] + g2i[0]) * third
    h = zu @ w1u[...] + zi @ w1i[...] + b1[...]
    h = jnp.maximum(h, 0.0)
    h = h @ w2[...] + b2[...]
    out[...] = h @ w3[...] + b3[...]


def _mlp(g0, g1, g2, w1u, w1i, b1, w2, b2, w3, b3, batch):
    bm = 1024
    grid = (batch // bm,)
    u_spec = lambda t: pl.BlockSpec((1, bm, EPAD), lambda i, _t=t: (_t, i, 0))
    w = lambda shape: pl.BlockSpec(shape, lambda i: (0,) * len(shape))
    return pl.pallas_call(
        _mlp_block,
        grid=grid,
        in_specs=[u_spec(0), u_spec(1), u_spec(0), u_spec(1),
                  u_spec(0), u_spec(1),
                  w((EPAD, 64)), w((EPAD, 64)), w((1, 64)), w((64, 32)),
                  w((1, 32)), w((32, 1)), w((1, 1))],
        out_specs=pl.BlockSpec((bm, 1), lambda i: (i, 0)),
        out_shape=jax.ShapeDtypeStruct((batch, 1), jnp.float32),
    )(g0, g0, g1, g1, g2, g2, w1u, w1i, b1, w2, b2, w3, b3)


def _to_chunk_major(side):
    side = jnp.pad(side, ((0, NP2 - side.shape[0]), (0, EPAD - EMB)))
    return side.reshape(NP2, NCH, CW).transpose(1, 0, 2)


def kernel(userIdx, itemIdx, adj_row, adj_col, adj_val,
           user_emb, item_emb, W1, b1, W2, b2, W3, b3):
    batch = userIdx.shape[0]
    e = adj_row.shape[0]
    eh = e // 2

    x0c = jnp.stack([_to_chunk_major(user_emb), _to_chunk_major(item_emb)])

    nbt = -(-eh // (NS * KB))
    nbh = nbt * NS
    ehp = nbh * KB

    def prep_side(rows, cols, vals, row_off, col_off):
        rows = jnp.pad(rows.astype(jnp.int32) - row_off, (0, ehp - eh))
        cols = jnp.pad(cols.astype(jnp.int32) - col_off, (0, ehp - eh))
        vals = jnp.pad(vals, (0, ehp - eh))
        return (jnp.stack([rows.reshape(-1, KB), cols.reshape(-1, KB)],
                          axis=1), vals.reshape(-1, KB))

    ebu, evu = prep_side(adj_row[:eh], adj_col[:eh], adj_val[:eh],
                         0, N_USERS)
    ebi, evi = prep_side(adj_row[eh:], adj_col[eh:], adj_val[eh:],
                         N_USERS, 0)
    ebat = jnp.concatenate([ebu, ebi])
    eval_ = jnp.concatenate([evu, evi])

    x1c = _layer(x0c, ebat, eval_, nbh)
    x2c = _layer(x1c, ebat, eval_, nbh)

    cidx = jnp.concatenate([userIdx.astype(jnp.int32),
                            itemIdx.astype(jnp.int32)])
    g0, g1, g2 = _batch_gather(x0c, x1c, x2c, cidx, 2 * batch)

    w1u = jnp.concatenate([W1[:EMB], jnp.zeros((EPAD - EMB, 64))])
    w1i = jnp.concatenate([W1[EMB:], jnp.zeros((EPAD - EMB, 64))])

    out = _mlp(g0.reshape(2, batch, EPAD), g1.reshape(2, batch, EPAD),
               g2.reshape(2, batch, EPAD),
               w1u.astype(jnp.float32), w1i.astype(jnp.float32),
               b1.reshape(1, 64), W2, b2.reshape(1, 32), W3,
               b3.reshape(1, 1), batch)
    return out.reshape(-1)
